# baseline (speedup 1.0000x reference)
import functools

import jax
import jax.numpy as jnp
from jax import lax
from jax.experimental import pallas as pl
from jax.experimental.pallas import tpu as pltpu
from jax.experimental.pallas import tpu_sc as plsc

B = 4096
V = 1000000
DIN = 64
DITEM = 128
NNEG = 20

NC = 2
NS = 16
NW = NC * NS
PB = B // NW
NB = PB * NNEG
CHUNK_B = 16
CHUNK_ROWS = CHUNK_B * NNEG
NUM_CHUNKS = PB // CHUNK_B

_sc_mesh = plsc.VectorSubcoreMesh(core_axis_name="c", subcore_axis_name="s")


def _tc_mlp_body(x_t_ref, w1t_ref, b1_ref, w2t_ref, b2_ref, out_ref):
    h = jnp.maximum(
        lax.dot_general(x_t_ref[...], w1t_ref[...], (((0,), (0,)), ((), ())),
                        preferred_element_type=jnp.float32) + b1_ref[...],
        0.0)
    u = (jnp.dot(h, w2t_ref[...], preferred_element_type=jnp.float32)
         + b2_ref[...])
    out_ref[...] = u


_tc_mlp = pl.pallas_call(
    _tc_mlp_body,
    out_shape=jax.ShapeDtypeStruct((B, DITEM), jnp.float32),
)


@functools.partial(
    pl.kernel,
    out_type=[
        jax.ShapeDtypeStruct((B, DITEM), jnp.float32),
        jax.ShapeDtypeStruct((B, DITEM), jnp.float32),
    ],
    mesh=_sc_mesh,
    compiler_params=pltpu.CompilerParams(needs_layout_passes=False),
    scratch_types=[
        pltpu.VMEM((PB,), jnp.int32),
        pltpu.VMEM((NB,), jnp.int32),
        pltpu.VMEM((PB, DITEM), jnp.float32),
        pltpu.VMEM((PB, DITEM), jnp.float32),
        pltpu.VMEM((CHUNK_ROWS, DITEM), jnp.float32),
        pltpu.VMEM((CHUNK_ROWS, DITEM), jnp.float32),
        pltpu.VMEM((32, 16), jnp.float32),
        pltpu.VMEM((CHUNK_B, DITEM), jnp.float32),
        pltpu.SemaphoreType.DMA,
        pltpu.SemaphoreType.DMA,
        pltpu.SemaphoreType.DMA,
    ],
)
def _sc_stage_c(pos_item_hbm, neg_idx_hbm, item_emb_hbm, emb_user_hbm,
                out_item_hbm, out_scores_hbm,
                idx_item_v, idx_v, rows_item_v, user_v, buf0, buf1, red_v,
                scores_v, sem0, sem1, sem_i):
    wid = lax.axis_index("s") * NC + lax.axis_index("c")
    base = wid * PB
    nbase = wid * NB
    pltpu.sync_copy(pos_item_hbm.at[pl.ds(base, PB)], idx_item_v)
    pltpu.sync_copy(neg_idx_hbm.at[pl.ds(nbase, NB)], idx_v)
    item_cp = pltpu.async_copy(item_emb_hbm.at[idx_item_v], rows_item_v,
                               sem_i)
    pltpu.sync_copy(emb_user_hbm.at[pl.ds(base, PB)], user_v)

    bufs = (buf0, buf1)
    sems = (sem0, sem1)

    def start(c):
        idx_slice = idx_v.at[pl.ds(c * CHUNK_ROWS, CHUNK_ROWS)]
        return pltpu.async_copy(item_emb_hbm.at[idx_slice], bufs[c % 2],
                                sems[c % 2])

    iota16 = lax.iota(jnp.int32, 16)
    pending = start(0)
    for c in range(NUM_CHUNKS):
        pending.wait()
        if c + 1 < NUM_CHUNKS:
            pending = start(c + 1)
        buf = bufs[c % 2]

        @pl.loop(0, CHUNK_B)
        def _batch_loop(j):
            row = c * CHUNK_B + j
            uv = [user_v[row, pl.ds(16 * k, 16)] for k in range(8)]
            for n in range(NNEG):
                r = j * NNEG + n
                acc = buf[r, pl.ds(0, 16)] * uv[0]
                for k in range(1, 8):
                    acc = acc + buf[r, pl.ds(16 * k, 16)] * uv[k]
                red_v[n, :] = acc
            res1 = plsc.load_gather(red_v, [iota16,
                                            jnp.zeros((16,), jnp.int32)])
            res2 = plsc.load_gather(red_v, [iota16 + 16,
                                            jnp.zeros((16,), jnp.int32)])
            for col in range(1, 16):
                cidx = jnp.full((16,), col, jnp.int32)
                res1 = res1 + plsc.load_gather(red_v, [iota16, cidx])
                res2 = res2 + plsc.load_gather(red_v, [iota16 + 16, cidx])
            scores_v[j, pl.ds(0, 16)] = res1
            scores_v[j, pl.ds(16, 16)] = res2

        pltpu.sync_copy(scores_v,
                        out_scores_hbm.at[pl.ds(base + c * CHUNK_B, CHUNK_B)])

    item_cp.wait()
    pltpu.sync_copy(rows_item_v, out_item_hbm.at[pl.ds(base, PB)])


def _tc_loss_body(user_ref, item_ref, scores_ref, out_ref):
    u = user_ref[...]
    s = jnp.sum(u * item_ref[...], axis=1)
    s = jnp.clip(s, -10.0, 10.0)
    pos_loss = jnp.log1p(jnp.exp(-s))
    ns = scores_ref[...][:, :NNEG]
    ns = jnp.clip(ns, -10.0, 10.0)
    neg_loss = jnp.sum(jnp.log1p(jnp.exp(ns)), axis=1)
    out_ref[...] = (jnp.sum(pos_loss + neg_loss) * (1.0 / B)).reshape(1, 1)


_tc_loss = pl.pallas_call(
    _tc_loss_body,
    out_shape=jax.ShapeDtypeStruct((1, 1), jnp.float32),
)


def kernel(pos_input, pos_item, neg_item, i, input_emb, item_emb, W1, b1, W2,
           b2):
    del i
    pos_input = pos_input.astype(jnp.int32)
    pos_item = pos_item.astype(jnp.int32)
    neg_flat = neg_item.reshape(B * NNEG).astype(jnp.int32)
    emb_in_t = jnp.take(input_emb.astype(jnp.bfloat16), pos_input,
                        axis=0).astype(jnp.float32).T
    emb_user = _tc_mlp(emb_in_t, W1.T, b1.reshape(1, 512), W2.T,
                       b2.reshape(1, DITEM))
    emb_item, scores = _sc_stage_c(pos_item, neg_flat, item_emb, emb_user)
    out = _tc_loss(emb_user, emb_item, scores)
    return out.reshape(())


# speedup vs baseline: 2.3615x; 2.3615x over previous
import functools

import jax
import jax.numpy as jnp
from jax import lax
from jax.experimental import pallas as pl
from jax.experimental.pallas import tpu as pltpu
from jax.experimental.pallas import tpu_sc as plsc

B = 4096
V = 1000000
DIN = 64
DITEM = 128
NNEG = 20

NC = 2
NS = 16
NW = NC * NS
PB = B // NW
NB = PB * NNEG
CHUNK_B = 16
CHUNK_ROWS = CHUNK_B * NNEG
NUM_CHUNKS = PB // CHUNK_B

_sc_mesh = plsc.VectorSubcoreMesh(core_axis_name="c", subcore_axis_name="s")


NRING = 8


@functools.partial(
    pl.kernel,
    out_type=jax.ShapeDtypeStruct((B, DIN), jnp.float32),
    mesh=_sc_mesh,
    compiler_params=pltpu.CompilerParams(needs_layout_passes=False,
                                         disable_bounds_checks=True),
    scratch_types=[
        pltpu.VMEM((PB,), jnp.int32),
        pltpu.VMEM((NRING, DIN, 128), jnp.float32),
        pltpu.VMEM((PB, DIN), jnp.float32),
    ] + [pltpu.SemaphoreType.DMA] * NRING,
)
def _sc_stage_a(pos_input_hbm, inp_t_hbm, out_hbm, idx_s, slabs,
                out_v, *sems):
    wid = lax.axis_index("s") * NC + lax.axis_index("c")
    base = wid * PB
    pltpu.sync_copy(pos_input_hbm.at[pl.ds(base, PB)], idx_s)
    iota16 = lax.iota(jnp.int32, 16)
    gvecs = [idx_s[pl.ds(16 * g, 16)] for g in range(PB // 16)]

    def getidx(j):
        return gvecs[j // 16][j % 16]

    def fire(j):
        c = lax.shift_right_logical(getidx(j), 7)
        coff = pl.multiple_of(c * 128, 128)
        return pltpu.async_copy(inp_t_hbm.at[:, pl.ds(coff, 128)],
                                slabs.at[j % NRING], sems[j % NRING])

    descs = [fire(j) for j in range(NRING)]
    for j in range(PB):
        descs[j % NRING].wait()
        lane = jnp.full((16,), getidx(j) & 127, jnp.int32)
        slab = slabs.at[j % NRING]
        for g in range(DIN // 16):
            vals = plsc.load_gather(slab, [iota16 + 16 * g, lane])
            out_v[j, pl.ds(16 * g, 16)] = vals
        if j + NRING < PB:
            descs[j % NRING] = fire(j + NRING)
    pltpu.sync_copy(out_v, out_hbm.at[pl.ds(base, PB)])


def _tc_mlp_body(x_ref, w1t_ref, b1_ref, w2t_ref, b2_ref, out_ref):
    h = jnp.maximum(
        jnp.dot(x_ref[...], w1t_ref[...],
                preferred_element_type=jnp.float32) + b1_ref[...],
        0.0)
    u = (jnp.dot(h, w2t_ref[...], preferred_element_type=jnp.float32)
         + b2_ref[...])
    out_ref[...] = u


_tc_mlp = pl.pallas_call(
    _tc_mlp_body,
    out_shape=jax.ShapeDtypeStruct((B, DITEM), jnp.float32),
)


@functools.partial(
    pl.kernel,
    out_type=[
        jax.ShapeDtypeStruct((B, DITEM), jnp.float32),
        jax.ShapeDtypeStruct((B, DITEM), jnp.float32),
    ],
    mesh=_sc_mesh,
    compiler_params=pltpu.CompilerParams(needs_layout_passes=False),
    scratch_types=[
        pltpu.VMEM((PB,), jnp.int32),
        pltpu.VMEM((NB,), jnp.int32),
        pltpu.VMEM((PB, DITEM), jnp.float32),
        pltpu.VMEM((PB, DITEM), jnp.float32),
        pltpu.VMEM((CHUNK_ROWS, DITEM), jnp.float32),
        pltpu.VMEM((CHUNK_ROWS, DITEM), jnp.float32),
        pltpu.VMEM((32, 16), jnp.float32),
        pltpu.VMEM((CHUNK_B, DITEM), jnp.float32),
        pltpu.SemaphoreType.DMA,
        pltpu.SemaphoreType.DMA,
        pltpu.SemaphoreType.DMA,
    ],
)
def _sc_stage_c(pos_item_hbm, neg_idx_hbm, item_emb_hbm, emb_user_hbm,
                out_item_hbm, out_scores_hbm,
                idx_item_v, idx_v, rows_item_v, user_v, buf0, buf1, red_v,
                scores_v, sem0, sem1, sem_i):
    wid = lax.axis_index("s") * NC + lax.axis_index("c")
    base = wid * PB
    nbase = wid * NB
    pltpu.sync_copy(pos_item_hbm.at[pl.ds(base, PB)], idx_item_v)
    pltpu.sync_copy(neg_idx_hbm.at[pl.ds(nbase, NB)], idx_v)
    item_cp = pltpu.async_copy(item_emb_hbm.at[idx_item_v], rows_item_v,
                               sem_i)
    pltpu.sync_copy(emb_user_hbm.at[pl.ds(base, PB)], user_v)

    bufs = (buf0, buf1)
    sems = (sem0, sem1)

    def start(c):
        idx_slice = idx_v.at[pl.ds(c * CHUNK_ROWS, CHUNK_ROWS)]
        return pltpu.async_copy(item_emb_hbm.at[idx_slice], bufs[c % 2],
                                sems[c % 2])

    iota16 = lax.iota(jnp.int32, 16)
    pending = start(0)
    for c in range(NUM_CHUNKS):
        pending.wait()
        if c + 1 < NUM_CHUNKS:
            pending = start(c + 1)
        buf = bufs[c % 2]

        @pl.loop(0, CHUNK_B)
        def _batch_loop(j):
            row = c * CHUNK_B + j
            uv = [user_v[row, pl.ds(16 * k, 16)] for k in range(8)]
            for n in range(NNEG):
                r = j * NNEG + n
                acc = buf[r, pl.ds(0, 16)] * uv[0]
                for k in range(1, 8):
                    acc = acc + buf[r, pl.ds(16 * k, 16)] * uv[k]
                red_v[n, :] = acc
            res1 = plsc.load_gather(red_v, [iota16,
                                            jnp.zeros((16,), jnp.int32)])
            res2 = plsc.load_gather(red_v, [iota16 + 16,
                                            jnp.zeros((16,), jnp.int32)])
            for col in range(1, 16):
                cidx = jnp.full((16,), col, jnp.int32)
                res1 = res1 + plsc.load_gather(red_v, [iota16, cidx])
                res2 = res2 + plsc.load_gather(red_v, [iota16 + 16, cidx])
            scores_v[j, pl.ds(0, 16)] = res1
            scores_v[j, pl.ds(16, 16)] = res2

        pltpu.sync_copy(scores_v,
                        out_scores_hbm.at[pl.ds(base + c * CHUNK_B, CHUNK_B)])

    item_cp.wait()
    pltpu.sync_copy(rows_item_v, out_item_hbm.at[pl.ds(base, PB)])


def _tc_loss_body(user_ref, item_ref, scores_ref, out_ref):
    u = user_ref[...]
    s = jnp.sum(u * item_ref[...], axis=1)
    s = jnp.clip(s, -10.0, 10.0)
    pos_loss = jnp.log1p(jnp.exp(-s))
    ns = scores_ref[...][:, :NNEG]
    ns = jnp.clip(ns, -10.0, 10.0)
    neg_loss = jnp.sum(jnp.log1p(jnp.exp(ns)), axis=1)
    out_ref[...] = (jnp.sum(pos_loss + neg_loss) * (1.0 / B)).reshape(1, 1)


_tc_loss = pl.pallas_call(
    _tc_loss_body,
    out_shape=jax.ShapeDtypeStruct((1, 1), jnp.float32),
)


def kernel(pos_input, pos_item, neg_item, i, input_emb, item_emb, W1, b1, W2,
           b2):
    del i
    pos_input = pos_input.astype(jnp.int32)
    pos_item = pos_item.astype(jnp.int32)
    neg_flat = neg_item.reshape(B * NNEG).astype(jnp.int32)
    emb_in = _sc_stage_a(pos_input, input_emb.T)
    emb_user = _tc_mlp(emb_in, W1.T, b1.reshape(1, 512), W2.T,
                       b2.reshape(1, DITEM))
    emb_item, scores = _sc_stage_c(pos_item, neg_flat, item_emb, emb_user)
    out = _tc_loss(emb_user, emb_item, scores)
    return out.reshape(())


# speedup vs baseline: 2.4734x; 1.0474x over previous
import functools

import jax
import jax.numpy as jnp
from jax import lax
from jax.experimental import pallas as pl
from jax.experimental.pallas import tpu as pltpu
from jax.experimental.pallas import tpu_sc as plsc

B = 4096
V = 1000000
DIN = 64
DITEM = 128
NNEG = 20

NC = 2
NS = 16
NW = NC * NS
PB = B // NW
NB = PB * NNEG
CHUNK_B = 16
CHUNK_ROWS = CHUNK_B * NNEG
NUM_CHUNKS = PB // CHUNK_B

_sc_mesh = plsc.VectorSubcoreMesh(core_axis_name="c", subcore_axis_name="s")


NRING = 8


@functools.partial(
    pl.kernel,
    out_type=jax.ShapeDtypeStruct((B, DIN), jnp.float32),
    mesh=_sc_mesh,
    compiler_params=pltpu.CompilerParams(needs_layout_passes=False,
                                         disable_bounds_checks=True),
    scratch_types=[
        pltpu.VMEM((PB,), jnp.int32),
        pltpu.VMEM((NRING, DIN, 128), jnp.float32),
        pltpu.VMEM((PB, DIN), jnp.float32),
    ] + [pltpu.SemaphoreType.DMA] * NRING,
)
def _sc_stage_a(pos_input_hbm, inp_t_hbm, out_hbm, idx_s, slabs,
                out_v, *sems):
    wid = lax.axis_index("s") * NC + lax.axis_index("c")
    base = wid * PB
    pltpu.sync_copy(pos_input_hbm.at[pl.ds(base, PB)], idx_s)
    iota16 = lax.iota(jnp.int32, 16)
    gvecs = [idx_s[pl.ds(16 * g, 16)] for g in range(PB // 16)]

    def getidx(j):
        return gvecs[j // 16][j % 16]

    def fire(j):
        c = lax.shift_right_logical(getidx(j), 7)
        coff = pl.multiple_of(c * 128, 128)
        return pltpu.async_copy(inp_t_hbm.at[:, pl.ds(coff, 128)],
                                slabs.at[j % NRING], sems[j % NRING])

    descs = [fire(j) for j in range(NRING)]
    for j in range(PB):
        descs[j % NRING].wait()
        lane = jnp.full((16,), getidx(j) & 127, jnp.int32)
        slab = slabs.at[j % NRING]
        for g in range(DIN // 16):
            vals = plsc.load_gather(slab, [iota16 + 16 * g, lane])
            out_v[j, pl.ds(16 * g, 16)] = vals
        if j + NRING < PB:
            descs[j % NRING] = fire(j + NRING)
    pltpu.sync_copy(out_v, out_hbm.at[pl.ds(base, PB)])


def _tc_mlp_body(x_ref, w1t_ref, b1_ref, w2t_ref, b2_ref, out_ref):
    h = jnp.maximum(
        jnp.dot(x_ref[...], w1t_ref[...],
                preferred_element_type=jnp.float32) + b1_ref[...],
        0.0)
    u = (jnp.dot(h, w2t_ref[...], preferred_element_type=jnp.float32)
         + b2_ref[...])
    out_ref[...] = u


_tc_mlp = pl.pallas_call(
    _tc_mlp_body,
    out_shape=jax.ShapeDtypeStruct((B, DITEM), jnp.float32),
)


@functools.partial(
    pl.kernel,
    out_type=[
        jax.ShapeDtypeStruct((B, DITEM), jnp.float32),
        jax.ShapeDtypeStruct((B, DITEM), jnp.float32),
    ],
    mesh=_sc_mesh,
    compiler_params=pltpu.CompilerParams(needs_layout_passes=False),
    scratch_types=[
        pltpu.VMEM((PB,), jnp.int32),
        pltpu.VMEM((NB,), jnp.int32),
        pltpu.VMEM((PB, DITEM), jnp.float32),
        pltpu.VMEM((PB, DITEM), jnp.float32),
        pltpu.VMEM((CHUNK_ROWS, DITEM), jnp.float32),
        pltpu.VMEM((CHUNK_ROWS, DITEM), jnp.float32),
        pltpu.VMEM((32, 16), jnp.float32),
        pltpu.VMEM((CHUNK_B, DITEM), jnp.float32),
        pltpu.SemaphoreType.DMA,
        pltpu.SemaphoreType.DMA,
        pltpu.SemaphoreType.DMA,
    ],
)
def _sc_stage_c(pos_item_hbm, neg_idx_hbm, item_emb_hbm, emb_user_hbm,
                out_item_hbm, out_scores_hbm,
                idx_item_v, idx_v, rows_item_v, user_v, buf0, buf1, red_v,
                scores_v, sem0, sem1, sem_i):
    wid = lax.axis_index("s") * NC + lax.axis_index("c")
    base = wid * PB
    nbase = wid * NB
    pltpu.sync_copy(pos_item_hbm.at[pl.ds(base, PB)], idx_item_v)
    pltpu.sync_copy(neg_idx_hbm.at[pl.ds(nbase, NB)], idx_v)
    item_cp = pltpu.async_copy(item_emb_hbm.at[idx_item_v], rows_item_v,
                               sem_i)
    pltpu.sync_copy(emb_user_hbm.at[pl.ds(base, PB)], user_v)

    bufs = (buf0, buf1)
    sems = (sem0, sem1)

    def start(c):
        idx_slice = idx_v.at[pl.ds(c * CHUNK_ROWS, CHUNK_ROWS)]
        return pltpu.async_copy(item_emb_hbm.at[idx_slice], bufs[c % 2],
                                sems[c % 2])

    iota16 = lax.iota(jnp.int32, 16)

    def tree_sum(vals):
        while len(vals) > 1:
            vals = [a + b for a, b in zip(vals[::2], vals[1::2])]
        return vals[0]

    pending = start(0)
    for c in range(NUM_CHUNKS):
        pending.wait()
        if c + 1 < NUM_CHUNKS:
            pending = start(c + 1)
        buf = bufs[c % 2]

        @pl.loop(0, CHUNK_B)
        def _batch_loop(j):
            row = c * CHUNK_B + j
            uv = [user_v[row, pl.ds(16 * k, 16)] for k in range(8)]
            for n in range(NNEG):
                r = j * NNEG + n
                red_v[n, :] = tree_sum(
                    [buf[r, pl.ds(16 * k, 16)] * uv[k] for k in range(8)])
            cols = [jnp.full((16,), col, jnp.int32) for col in range(16)]
            res1 = tree_sum(
                [plsc.load_gather(red_v, [iota16, ci]) for ci in cols])
            res2 = tree_sum(
                [plsc.load_gather(red_v, [iota16 + 16, ci]) for ci in cols])
            scores_v[j, pl.ds(0, 16)] = res1
            scores_v[j, pl.ds(16, 16)] = res2

        pltpu.sync_copy(scores_v,
                        out_scores_hbm.at[pl.ds(base + c * CHUNK_B, CHUNK_B)])

    item_cp.wait()
    pltpu.sync_copy(rows_item_v, out_item_hbm.at[pl.ds(base, PB)])


def _tc_loss_body(user_ref, item_ref, scores_ref, out_ref):
    u = user_ref[...]
    s = jnp.sum(u * item_ref[...], axis=1)
    s = jnp.clip(s, -10.0, 10.0)
    pos_loss = jnp.log1p(jnp.exp(-s))
    ns = scores_ref[...][:, :NNEG]
    ns = jnp.clip(ns, -10.0, 10.0)
    neg_loss = jnp.sum(jnp.log1p(jnp.exp(ns)), axis=1)
    out_ref[...] = (jnp.sum(pos_loss + neg_loss) * (1.0 / B)).reshape(1, 1)


_tc_loss = pl.pallas_call(
    _tc_loss_body,
    out_shape=jax.ShapeDtypeStruct((1, 1), jnp.float32),
)


def kernel(pos_input, pos_item, neg_item, i, input_emb, item_emb, W1, b1, W2,
           b2):
    del i
    pos_input = pos_input.astype(jnp.int32)
    pos_item = pos_item.astype(jnp.int32)
    neg_flat = neg_item.reshape(B * NNEG).astype(jnp.int32)
    emb_in = _sc_stage_a(pos_input, input_emb.T)
    emb_user = _tc_mlp(emb_in, W1.T, b1.reshape(1, 512), W2.T,
                       b2.reshape(1, DITEM))
    emb_item, scores = _sc_stage_c(pos_item, neg_flat, item_emb, emb_user)
    out = _tc_loss(emb_user, emb_item, scores)
    return out.reshape(())


# speedup vs baseline: 2.8307x; 1.1445x over previous
import functools

import jax
import jax.numpy as jnp
from jax import lax
from jax.experimental import pallas as pl
from jax.experimental.pallas import tpu as pltpu
from jax.experimental.pallas import tpu_sc as plsc

B = 4096
V = 1000000
DIN = 64
DITEM = 128
NNEG = 20

NC = 2
NS = 16
NW = NC * NS
PB = B // NW
NB = PB * NNEG
CHUNK_B = 16
CHUNK_ROWS = CHUNK_B * NNEG
NUM_CHUNKS = PB // CHUNK_B

_sc_mesh = plsc.VectorSubcoreMesh(core_axis_name="c", subcore_axis_name="s")


NRING = 8


@functools.partial(
    pl.kernel,
    out_type=jax.ShapeDtypeStruct((B, DIN), jnp.float32),
    mesh=_sc_mesh,
    compiler_params=pltpu.CompilerParams(needs_layout_passes=False,
                                         disable_bounds_checks=True),
    scratch_types=[
        pltpu.VMEM((PB,), jnp.int32),
        pltpu.VMEM((NRING, DIN, 128), jnp.float32),
        pltpu.VMEM((PB, DIN), jnp.float32),
    ] + [pltpu.SemaphoreType.DMA] * NRING,
)
def _sc_stage_a(pos_input_hbm, inp_t_hbm, out_hbm, idx_s, slabs,
                out_v, *sems):
    wid = lax.axis_index("s") * NC + lax.axis_index("c")
    base = wid * PB
    pltpu.sync_copy(pos_input_hbm.at[pl.ds(base, PB)], idx_s)
    iota16 = lax.iota(jnp.int32, 16)
    gvecs = [idx_s[pl.ds(16 * g, 16)] for g in range(PB // 16)]

    def getidx(j):
        return gvecs[j // 16][j % 16]

    def fire(j):
        c = lax.shift_right_logical(getidx(j), 7)
        coff = pl.multiple_of(c * 128, 128)
        return pltpu.async_copy(inp_t_hbm.at[:, pl.ds(coff, 128)],
                                slabs.at[j % NRING], sems[j % NRING])

    descs = [fire(j) for j in range(NRING)]
    for j in range(PB):
        descs[j % NRING].wait()
        lane = jnp.full((16,), getidx(j) & 127, jnp.int32)
        slab = slabs.at[j % NRING]
        for g in range(DIN // 16):
            vals = plsc.load_gather(slab, [iota16 + 16 * g, lane])
            out_v[j, pl.ds(16 * g, 16)] = vals
        if j + NRING < PB:
            descs[j % NRING] = fire(j + NRING)
    pltpu.sync_copy(out_v, out_hbm.at[pl.ds(base, PB)])


def _tc_mlp_body(x_ref, w1t_ref, b1_ref, w2t_ref, b2_ref, out_ref):
    h = jnp.maximum(
        jnp.dot(x_ref[...], w1t_ref[...],
                preferred_element_type=jnp.float32) + b1_ref[...],
        0.0)
    u = (jnp.dot(h, w2t_ref[...], preferred_element_type=jnp.float32)
         + b2_ref[...])
    out_ref[...] = u


_tc_mlp = pl.pallas_call(
    _tc_mlp_body,
    out_shape=jax.ShapeDtypeStruct((B, DITEM), jnp.float32),
)


@functools.partial(
    pl.kernel,
    out_type=[
        jax.ShapeDtypeStruct((B, DITEM), jnp.float32),
        jax.ShapeDtypeStruct((B * NNEG // 8, DITEM), jnp.float32),
    ],
    mesh=_sc_mesh,
    compiler_params=pltpu.CompilerParams(needs_layout_passes=False),
    scratch_types=[
        pltpu.VMEM((PB,), jnp.int32),
        pltpu.VMEM((NB,), jnp.int32),
        pltpu.VMEM((PB, DITEM), jnp.float32),
        pltpu.VMEM((PB, DITEM), jnp.float32),
        pltpu.VMEM((CHUNK_ROWS, DITEM), jnp.float32),
        pltpu.VMEM((CHUNK_ROWS, DITEM), jnp.float32),
        pltpu.VMEM((CHUNK_ROWS // 8, DITEM), jnp.float32),
        pltpu.SemaphoreType.DMA,
        pltpu.SemaphoreType.DMA,
        pltpu.SemaphoreType.DMA,
    ],
)
def _sc_stage_c(pos_item_hbm, neg_idx_hbm, item_emb_hbm, emb_user_hbm,
                out_item_hbm, out_scores_hbm,
                idx_item_v, idx_v, rows_item_v, user_v, buf0, buf1,
                pbuf, sem0, sem1, sem_i):
    wid = lax.axis_index("s") * NC + lax.axis_index("c")
    base = wid * PB
    nbase = wid * NB
    pltpu.sync_copy(pos_item_hbm.at[pl.ds(base, PB)], idx_item_v)
    pltpu.sync_copy(neg_idx_hbm.at[pl.ds(nbase, NB)], idx_v)
    item_cp = pltpu.async_copy(item_emb_hbm.at[idx_item_v], rows_item_v,
                               sem_i)
    pltpu.sync_copy(emb_user_hbm.at[pl.ds(base, PB)], user_v)

    bufs = (buf0, buf1)
    sems = (sem0, sem1)

    def start(c):
        idx_slice = idx_v.at[pl.ds(c * CHUNK_ROWS, CHUNK_ROWS)]
        return pltpu.async_copy(item_emb_hbm.at[idx_slice], bufs[c % 2],
                                sems[c % 2])

    def tree_sum(vals):
        while len(vals) > 1:
            vals = [a + b for a, b in zip(vals[::2], vals[1::2])]
        return vals[0]

    prow_base = wid * (NB // 8)
    pending = start(0)
    for c in range(NUM_CHUNKS):
        pending.wait()
        if c + 1 < NUM_CHUNKS:
            pending = start(c + 1)
        buf = bufs[c % 2]

        @pl.loop(0, CHUNK_B // 2)
        def _batch_loop(t):
            for half in range(2):
                j = t * 2 + half
                row = c * CHUNK_B + j
                uv = [user_v[row, pl.ds(16 * k, 16)] for k in range(8)]
                for n in range(NNEG):
                    r = j * NNEG + n
                    s = tree_sum([buf[r, pl.ds(16 * k, 16)] * uv[k]
                                  for k in range(8)])
                    prow = t * 5 + (half * NNEG + n) // 8
                    poff = 16 * ((half * NNEG + n) % 8)
                    pbuf[prow, pl.ds(poff, 16)] = s

        pltpu.sync_copy(
            pbuf,
            out_scores_hbm.at[pl.ds(prow_base + c * (CHUNK_ROWS // 8),
                                    CHUNK_ROWS // 8)])

    item_cp.wait()
    pltpu.sync_copy(rows_item_v, out_item_hbm.at[pl.ds(base, PB)])


def _tc_loss_body(user_ref, item_ref, part_ref, out_ref):
    u = user_ref[...]
    s = jnp.sum(u * item_ref[...], axis=1)
    s = jnp.clip(s, -10.0, 10.0)
    pos_loss = jnp.sum(jnp.log1p(jnp.exp(-s)))
    r_iota = lax.broadcasted_iota(jnp.int32, (DITEM, 8), 0)
    c_iota = lax.broadcasted_iota(jnp.int32, (DITEM, 8), 1)
    blk = (r_iota // 16 == c_iota).astype(jnp.float32)
    ns = jnp.dot(part_ref[...], blk, preferred_element_type=jnp.float32)
    ns = jnp.clip(ns, -10.0, 10.0)
    neg_loss = jnp.sum(jnp.log1p(jnp.exp(ns)))
    out_ref[...] = ((pos_loss + neg_loss) * (1.0 / B)).reshape(1, 1)


_tc_loss = pl.pallas_call(
    _tc_loss_body,
    out_shape=jax.ShapeDtypeStruct((1, 1), jnp.float32),
)


def kernel(pos_input, pos_item, neg_item, i, input_emb, item_emb, W1, b1, W2,
           b2):
    del i
    pos_input = pos_input.astype(jnp.int32)
    pos_item = pos_item.astype(jnp.int32)
    neg_flat = neg_item.reshape(B * NNEG).astype(jnp.int32)
    emb_in = _sc_stage_a(pos_input, input_emb.T)
    emb_user = _tc_mlp(emb_in, W1.T, b1.reshape(1, 512), W2.T,
                       b2.reshape(1, DITEM))
    emb_item, scores = _sc_stage_c(pos_item, neg_flat, item_emb, emb_user)
    out = _tc_loss(emb_user, emb_item, scores)
    return out.reshape(())


# speedup vs baseline: 2.9662x; 1.0479x over previous
import functools

import jax
import jax.numpy as jnp
from jax import lax
from jax.experimental import pallas as pl
from jax.experimental.pallas import tpu as pltpu
from jax.experimental.pallas import tpu_sc as plsc

B = 4096
V = 1000000
DIN = 64
DITEM = 128
NNEG = 20

NC = 2
NS = 16
NW = NC * NS
PB = B // NW
NB = PB * NNEG
CHUNK_B = 16
CHUNK_ROWS = CHUNK_B * NNEG
NUM_CHUNKS = PB // CHUNK_B

_sc_mesh = plsc.VectorSubcoreMesh(core_axis_name="c", subcore_axis_name="s")


NRING = 8


@functools.partial(
    pl.kernel,
    out_type=jax.ShapeDtypeStruct((B, DIN), jnp.float32),
    mesh=_sc_mesh,
    compiler_params=pltpu.CompilerParams(needs_layout_passes=False,
                                         disable_bounds_checks=True),
    scratch_types=[
        pltpu.VMEM((PB,), jnp.int32),
        pltpu.VMEM((NRING, DIN, 128), jnp.float32),
        pltpu.VMEM((PB, DIN), jnp.float32),
    ] + [pltpu.SemaphoreType.DMA] * NRING,
)
def _sc_stage_a(pos_input_hbm, inp_t_hbm, out_hbm, idx_s, slabs,
                out_v, *sems):
    wid = lax.axis_index("s") * NC + lax.axis_index("c")
    base = wid * PB
    pltpu.sync_copy(pos_input_hbm.at[pl.ds(base, PB)], idx_s)
    iota16 = lax.iota(jnp.int32, 16)
    gvecs = [idx_s[pl.ds(16 * g, 16)] for g in range(PB // 16)]

    def getidx(j):
        return gvecs[j // 16][j % 16]

    def fire(j):
        c = lax.shift_right_logical(getidx(j), 7)
        coff = pl.multiple_of(c * 128, 128)
        return pltpu.async_copy(inp_t_hbm.at[:, pl.ds(coff, 128)],
                                slabs.at[j % NRING], sems[j % NRING])

    descs = [fire(j) for j in range(NRING)]
    for j in range(PB):
        descs[j % NRING].wait()
        lane = jnp.full((16,), getidx(j) & 127, jnp.int32)
        slab = slabs.at[j % NRING]
        for g in range(DIN // 16):
            vals = plsc.load_gather(slab, [iota16 + 16 * g, lane])
            out_v[j, pl.ds(16 * g, 16)] = vals
        if j + NRING < PB:
            descs[j % NRING] = fire(j + NRING)
    pltpu.sync_copy(out_v, out_hbm.at[pl.ds(base, PB)])


def _tc_mlp_body(x_ref, w1t_ref, b1_ref, w2t_ref, b2_ref, out_ref):
    h = jnp.maximum(
        jnp.dot(x_ref[...], w1t_ref[...],
                preferred_element_type=jnp.float32) + b1_ref[...],
        0.0)
    u = (jnp.dot(h, w2t_ref[...], preferred_element_type=jnp.float32)
         + b2_ref[...])
    out_ref[...] = u


_tc_mlp = pl.pallas_call(
    _tc_mlp_body,
    out_shape=jax.ShapeDtypeStruct((B, DITEM), jnp.float32),
)


@functools.partial(
    pl.kernel,
    out_type=[
        jax.ShapeDtypeStruct((B, DITEM), jnp.float32),
        jax.ShapeDtypeStruct((B * NNEG // 8, DITEM), jnp.float32),
    ],
    mesh=_sc_mesh,
    compiler_params=pltpu.CompilerParams(needs_layout_passes=False),
    scratch_types=[
        pltpu.VMEM((PB,), jnp.int32),
        pltpu.VMEM((NB,), jnp.int32),
        pltpu.VMEM((PB, DITEM), jnp.float32),
        pltpu.VMEM((PB, DITEM), jnp.float32),
        pltpu.VMEM((CHUNK_ROWS, DITEM), jnp.float32),
        pltpu.VMEM((CHUNK_ROWS, DITEM), jnp.float32),
        pltpu.VMEM((CHUNK_ROWS // 8, DITEM), jnp.float32),
        pltpu.SemaphoreType.DMA,
        pltpu.SemaphoreType.DMA,
        pltpu.SemaphoreType.DMA,
    ],
)
def _sc_stage_c(pos_item_hbm, neg_idx_hbm, item_emb_hbm, emb_user_hbm,
                out_item_hbm, out_scores_hbm,
                idx_item_v, idx_v, rows_item_v, user_v, buf0, buf1,
                pbuf, sem0, sem1, sem_i):
    wid = lax.axis_index("s") * NC + lax.axis_index("c")
    base = wid * PB
    nbase = wid * NB
    pltpu.sync_copy(neg_idx_hbm.at[pl.ds(nbase, NB)], idx_v)

    bufs = (buf0, buf1)
    sems = (sem0, sem1)

    def fire(c, slot):
        coff = pl.multiple_of(c * CHUNK_ROWS, 8)
        idx_slice = idx_v.at[pl.ds(coff, CHUNK_ROWS)]
        pltpu.async_copy(item_emb_hbm.at[idx_slice], bufs[slot], sems[slot])

    fire(0, 0)
    pltpu.sync_copy(pos_item_hbm.at[pl.ds(base, PB)], idx_item_v)
    item_cp = pltpu.async_copy(item_emb_hbm.at[idx_item_v], rows_item_v,
                               sem_i)
    pltpu.sync_copy(emb_user_hbm.at[pl.ds(base, PB)], user_v)

    def tree_sum(vals):
        while len(vals) > 1:
            vals = [a + b for a, b in zip(vals[::2], vals[1::2])]
        return vals[0]

    prow_base = wid * (NB // 8)

    @pl.loop(0, NUM_CHUNKS, step=2)
    def _chunk_loop(c0):
        for h in range(2):
            c = c0 + h
            pltpu.make_async_copy(item_emb_hbm.at[pl.ds(0, CHUNK_ROWS)],
                                  bufs[h], sems[h]).wait()

            @pl.when(c + 1 < NUM_CHUNKS)
            def _prefetch():
                fire(c + 1, 1 - h)

            buf = bufs[h]

            @pl.loop(0, CHUNK_B // 2)
            def _batch_loop(t):
                for half in range(2):
                    j = t * 2 + half
                    row = c * CHUNK_B + j
                    uv = [user_v[row, pl.ds(16 * k, 16)] for k in range(8)]
                    for n in range(NNEG):
                        r = j * NNEG + n
                        s = tree_sum([buf[r, pl.ds(16 * k, 16)] * uv[k]
                                      for k in range(8)])
                        prow = t * 5 + (half * NNEG + n) // 8
                        poff = 16 * ((half * NNEG + n) % 8)
                        pbuf[prow, pl.ds(poff, 16)] = s

            wb = pl.multiple_of(prow_base + c * (CHUNK_ROWS // 8), 8)
            pltpu.sync_copy(pbuf,
                            out_scores_hbm.at[pl.ds(wb, CHUNK_ROWS // 8)])

    item_cp.wait()
    pltpu.sync_copy(rows_item_v, out_item_hbm.at[pl.ds(base, PB)])


def _tc_loss_body(user_ref, item_ref, part_ref, out_ref):
    u = user_ref[...]
    s = jnp.sum(u * item_ref[...], axis=1)
    s = jnp.clip(s, -10.0, 10.0)
    pos_loss = jnp.sum(jnp.log1p(jnp.exp(-s)))
    r_iota = lax.broadcasted_iota(jnp.int32, (DITEM, 8), 0)
    c_iota = lax.broadcasted_iota(jnp.int32, (DITEM, 8), 1)
    blk = (r_iota // 16 == c_iota).astype(jnp.float32)
    ns = jnp.dot(part_ref[...], blk, preferred_element_type=jnp.float32)
    ns = jnp.clip(ns, -10.0, 10.0)
    neg_loss = jnp.sum(jnp.log1p(jnp.exp(ns)))
    out_ref[...] = ((pos_loss + neg_loss) * (1.0 / B)).reshape(1, 1)


_tc_loss = pl.pallas_call(
    _tc_loss_body,
    out_shape=jax.ShapeDtypeStruct((1, 1), jnp.float32),
)


def kernel(pos_input, pos_item, neg_item, i, input_emb, item_emb, W1, b1, W2,
           b2):
    del i
    pos_input = pos_input.astype(jnp.int32)
    pos_item = pos_item.astype(jnp.int32)
    neg_flat = neg_item.reshape(B * NNEG).astype(jnp.int32)
    emb_in = _sc_stage_a(pos_input, input_emb.T)
    emb_user = _tc_mlp(emb_in, W1.T, b1.reshape(1, 512), W2.T,
                       b2.reshape(1, DITEM))
    emb_item, scores = _sc_stage_c(pos_item, neg_flat, item_emb, emb_user)
    out = _tc_loss(emb_user, emb_item, scores)
    return out.reshape(())


# speedup vs baseline: 2.9743x; 1.0027x over previous
import functools

import jax
import jax.numpy as jnp
from jax import lax
from jax.experimental import pallas as pl
from jax.experimental.pallas import tpu as pltpu
from jax.experimental.pallas import tpu_sc as plsc

B = 4096
V = 1000000
DIN = 64
DITEM = 128
NNEG = 20

NC = 2
NS = 16
NW = NC * NS
PB = B // NW
NB = PB * NNEG
CHUNK_B = 16
CHUNK_ROWS = CHUNK_B * NNEG
NUM_CHUNKS = PB // CHUNK_B

_sc_mesh = plsc.VectorSubcoreMesh(core_axis_name="c", subcore_axis_name="s")


NRING = 8


@functools.partial(
    pl.kernel,
    out_type=jax.ShapeDtypeStruct((B, DIN), jnp.float32),
    mesh=_sc_mesh,
    compiler_params=pltpu.CompilerParams(needs_layout_passes=False,
                                         disable_bounds_checks=True),
    scratch_types=[
        pltpu.VMEM((PB,), jnp.int32),
        pltpu.VMEM((NRING, DIN, 128), jnp.float32),
        pltpu.VMEM((PB, DIN), jnp.float32),
    ] + [pltpu.SemaphoreType.DMA] * NRING,
)
def _sc_stage_a(pos_input_hbm, inp_t_hbm, out_hbm, idx_s, slabs,
                out_v, *sems):
    wid = lax.axis_index("s") * NC + lax.axis_index("c")
    base = wid * PB
    pltpu.sync_copy(pos_input_hbm.at[pl.ds(base, PB)], idx_s)
    iota16 = lax.iota(jnp.int32, 16)
    gvecs = [idx_s[pl.ds(16 * g, 16)] for g in range(PB // 16)]

    def getidx(j):
        return gvecs[j // 16][j % 16]

    def fire(j):
        c = lax.shift_right_logical(getidx(j), 7)
        coff = pl.multiple_of(c * 128, 128)
        return pltpu.async_copy(inp_t_hbm.at[:, pl.ds(coff, 128)],
                                slabs.at[j % NRING], sems[j % NRING])

    descs = [fire(j) for j in range(NRING)]
    for j in range(PB):
        descs[j % NRING].wait()
        lane = jnp.full((16,), getidx(j) & 127, jnp.int32)
        slab = slabs.at[j % NRING]
        for g in range(DIN // 16):
            vals = plsc.load_gather(slab, [iota16 + 16 * g, lane])
            out_v[j, pl.ds(16 * g, 16)] = vals
        if j + NRING < PB:
            descs[j % NRING] = fire(j + NRING)
    pltpu.sync_copy(out_v, out_hbm.at[pl.ds(base, PB)])


def _tc_mlp_body(x_ref, w1t_ref, b1_ref, w2t_ref, b2_ref, out_ref):
    x16 = x_ref[...].astype(jnp.bfloat16)
    w116 = w1t_ref[...].astype(jnp.bfloat16)
    h = jnp.maximum(
        jnp.dot(x16, w116, preferred_element_type=jnp.float32)
        + b1_ref[...], 0.0)
    u = (jnp.dot(h.astype(jnp.bfloat16),
                 w2t_ref[...].astype(jnp.bfloat16),
                 preferred_element_type=jnp.float32)
         + b2_ref[...])
    out_ref[...] = u


_tc_mlp = pl.pallas_call(
    _tc_mlp_body,
    out_shape=jax.ShapeDtypeStruct((B, DITEM), jnp.float32),
)


@functools.partial(
    pl.kernel,
    out_type=[
        jax.ShapeDtypeStruct((B, DITEM), jnp.float32),
        jax.ShapeDtypeStruct((B * NNEG // 8, DITEM), jnp.float32),
    ],
    mesh=_sc_mesh,
    compiler_params=pltpu.CompilerParams(needs_layout_passes=False),
    scratch_types=[
        pltpu.VMEM((PB,), jnp.int32),
        pltpu.VMEM((NB,), jnp.int32),
        pltpu.VMEM((PB, DITEM), jnp.float32),
        pltpu.VMEM((PB, DITEM), jnp.float32),
        pltpu.VMEM((CHUNK_ROWS, DITEM), jnp.float32),
        pltpu.VMEM((CHUNK_ROWS, DITEM), jnp.float32),
        pltpu.VMEM((CHUNK_ROWS // 8, DITEM), jnp.float32),
        pltpu.SemaphoreType.DMA,
        pltpu.SemaphoreType.DMA,
        pltpu.SemaphoreType.DMA,
    ],
)
def _sc_stage_c(pos_item_hbm, neg_idx_hbm, item_emb_hbm, emb_user_hbm,
                out_item_hbm, out_scores_hbm,
                idx_item_v, idx_v, rows_item_v, user_v, buf0, buf1,
                pbuf, sem0, sem1, sem_i):
    wid = lax.axis_index("s") * NC + lax.axis_index("c")
    base = wid * PB
    nbase = wid * NB
    pltpu.sync_copy(neg_idx_hbm.at[pl.ds(nbase, NB)], idx_v)

    bufs = (buf0, buf1)
    sems = (sem0, sem1)

    def fire(c, slot):
        coff = pl.multiple_of(c * CHUNK_ROWS, 8)
        idx_slice = idx_v.at[pl.ds(coff, CHUNK_ROWS)]
        pltpu.async_copy(item_emb_hbm.at[idx_slice], bufs[slot], sems[slot])

    fire(0, 0)
    pltpu.sync_copy(pos_item_hbm.at[pl.ds(base, PB)], idx_item_v)
    item_cp = pltpu.async_copy(item_emb_hbm.at[idx_item_v], rows_item_v,
                               sem_i)
    pltpu.sync_copy(emb_user_hbm.at[pl.ds(base, PB)], user_v)

    def tree_sum(vals):
        while len(vals) > 1:
            vals = [a + b for a, b in zip(vals[::2], vals[1::2])]
        return vals[0]

    prow_base = wid * (NB // 8)

    @pl.loop(0, NUM_CHUNKS, step=2)
    def _chunk_loop(c0):
        for h in range(2):
            c = c0 + h
            pltpu.make_async_copy(item_emb_hbm.at[pl.ds(0, CHUNK_ROWS)],
                                  bufs[h], sems[h]).wait()

            @pl.when(c + 1 < NUM_CHUNKS)
            def _prefetch():
                fire(c + 1, 1 - h)

            buf = bufs[h]

            @pl.loop(0, CHUNK_B // 2)
            def _batch_loop(t):
                for half in range(2):
                    j = t * 2 + half
                    row = c * CHUNK_B + j
                    uv = [user_v[row, pl.ds(16 * k, 16)] for k in range(8)]
                    for n in range(NNEG):
                        r = j * NNEG + n
                        s = tree_sum([buf[r, pl.ds(16 * k, 16)] * uv[k]
                                      for k in range(8)])
                        prow = t * 5 + (half * NNEG + n) // 8
                        poff = 16 * ((half * NNEG + n) % 8)
                        pbuf[prow, pl.ds(poff, 16)] = s

            wb = pl.multiple_of(prow_base + c * (CHUNK_ROWS // 8), 8)
            pltpu.sync_copy(pbuf,
                            out_scores_hbm.at[pl.ds(wb, CHUNK_ROWS // 8)])

    item_cp.wait()
    pltpu.sync_copy(rows_item_v, out_item_hbm.at[pl.ds(base, PB)])


def _tc_loss_body(user_ref, item_ref, part_ref, out_ref):
    u = user_ref[...]
    s = jnp.sum(u * item_ref[...], axis=1)
    s = jnp.clip(s, -10.0, 10.0)
    pos_loss = jnp.sum(jnp.log1p(jnp.exp(-s)))
    r_iota = lax.broadcasted_iota(jnp.int32, (DITEM, 8), 0)
    c_iota = lax.broadcasted_iota(jnp.int32, (DITEM, 8), 1)
    blk = (r_iota // 16 == c_iota).astype(jnp.float32)
    ns = jnp.dot(part_ref[...], blk, preferred_element_type=jnp.float32)
    ns = jnp.clip(ns, -10.0, 10.0)
    neg_loss = jnp.sum(jnp.log1p(jnp.exp(ns)))
    out_ref[...] = ((pos_loss + neg_loss) * (1.0 / B)).reshape(1, 1)


_tc_loss = pl.pallas_call(
    _tc_loss_body,
    out_shape=jax.ShapeDtypeStruct((1, 1), jnp.float32),
)


def kernel(pos_input, pos_item, neg_item, i, input_emb, item_emb, W1, b1, W2,
           b2):
    del i
    pos_input = pos_input.astype(jnp.int32)
    pos_item = pos_item.astype(jnp.int32)
    neg_flat = neg_item.reshape(B * NNEG).astype(jnp.int32)
    emb_in = _sc_stage_a(pos_input, input_emb.T)
    emb_user = _tc_mlp(emb_in, W1.T, b1.reshape(1, 512), W2.T,
                       b2.reshape(1, DITEM))
    emb_item, scores = _sc_stage_c(pos_item, neg_flat, item_emb, emb_user)
    out = _tc_loss(emb_user, emb_item, scores)
    return out.reshape(())
